# baseline (speedup 1.0000x reference)
import functools

import jax
import jax.numpy as jnp
from jax import lax
from jax.experimental import pallas as pl
from jax.experimental.pallas import tpu as pltpu
from jax.experimental.pallas import tpu_sc as plsc

N = 100000
D = 128
S = 256
R = 2000
NB = N // R
NEG = -1e30

NC, NS, NW = 2, 16, 32
T = 400
SB = 80
NSB = T // SB
NCHUNK = N // T
CPW = (NCHUNK + NW - 1) // NW
CW = 128

_INTERPRET = False


def _onehot(seg, dtype=jnp.float32):
    cols = lax.broadcasted_iota(jnp.int32, (R, S), 1)
    return jnp.where(seg[:, None] == cols, jnp.array(1.0, dtype), jnp.array(0.0, dtype))


def _pass1_body(xa_ref, sa_ref, xb_ref, sb_ref, sums_ref, cnts_ref):
    i = pl.program_id(0)

    @pl.when(i == 0)
    def _():
        sums_ref[...] = jnp.zeros_like(sums_ref)
        cnts_ref[...] = jnp.zeros_like(cnts_ref)

    for side, (x_ref, s_ref) in enumerate(((xa_ref, sa_ref), (xb_ref, sb_ref))):
        seg = s_ref[0, 0, :]
        oh = _onehot(seg)
        x = x_ref[...]
        sums_ref[side] += lax.dot_general(oh, x, (((0,), (0,)), ((), ())),
                                          preferred_element_type=jnp.float32)
        cnts_ref[side] += jnp.sum(oh, axis=0)


def _sc_pass1_body(xa_hbm, sa_hbm, xb_hbm, sb_hbm, sums_out, cnts_out,
                   xbuf, idxbuf, zbuf, ones_buf,
                   tab_a, tab_b, cnt_a, cnt_b):
    cid = lax.axis_index("c")
    sid = lax.axis_index("s")
    w = sid * NC + cid

    one = jnp.ones((16,), jnp.float32)
    zero = jnp.zeros((16,), jnp.float32)
    for r in range(16):
        for j in range(D // 16):
            zbuf[r, pl.ds(16 * j, 16)] = zero
    for r in range(SB):
        for j in range(CW // 16):
            ones_buf[r, pl.ds(16 * j, 16)] = one

    row0 = sid * 16
    pltpu.sync_copy(zbuf, tab_a.at[pl.ds(row0, 16)])
    pltpu.sync_copy(zbuf, tab_b.at[pl.ds(row0, 16)])
    pltpu.sync_copy(zbuf, cnt_a.at[pl.ds(row0, 16)])
    pltpu.sync_copy(zbuf, cnt_b.at[pl.ds(row0, 16)])
    plsc.subcore_barrier()

    for x_hbm, s_hbm, tab, cnt in ((xa_hbm, sa_hbm, tab_a, cnt_a),
                                   (xb_hbm, sb_hbm, tab_b, cnt_b)):
        for j in range(CPW):
            k = w + NW * j

            @pl.when(k < NCHUNK)
            def _():
                off = k * T
                pltpu.sync_copy(x_hbm.at[pl.ds(off, T)], xbuf)
                for i in range(NSB):
                    pltpu.sync_copy(s_hbm.at[pl.ds(off + i * SB, SB)],
                                    idxbuf.at[i])
                for i in range(NSB):
                    pltpu.sync_copy(xbuf.at[pl.ds(i * SB, SB)],
                                    tab.at[idxbuf.at[i]], add=True)
                    pltpu.sync_copy(ones_buf, cnt.at[idxbuf.at[i]], add=True)

    plsc.subcore_barrier()

    @pl.when(sid == 0)
    def _():
        pltpu.sync_copy(tab_a, sums_out.at[0, cid])
        pltpu.sync_copy(tab_b, sums_out.at[1, cid])
        pltpu.sync_copy(cnt_a, cnts_out.at[0, cid])
        pltpu.sync_copy(cnt_b, cnts_out.at[1, cid])


def _sc_pass1(x_A, segA_i32, x_B, segB_i32):
    return pl.kernel(
        _sc_pass1_body,
        out_type=[jax.ShapeDtypeStruct((2, NC, S, D), jnp.float32),
                  jax.ShapeDtypeStruct((2, NC, S, CW), jnp.float32)],
        mesh=plsc.VectorSubcoreMesh(core_axis_name="c", subcore_axis_name="s"),
        scratch_types=[
            pltpu.VMEM((T, D), jnp.float32),
            pltpu.VMEM((NSB, SB), jnp.int32),
            pltpu.VMEM((16, D), jnp.float32),
            pltpu.VMEM((SB, CW), jnp.float32),
            pltpu.VMEM_SHARED((S, D), jnp.float32),
            pltpu.VMEM_SHARED((S, D), jnp.float32),
            pltpu.VMEM_SHARED((S, CW), jnp.float32),
            pltpu.VMEM_SHARED((S, CW), jnp.float32),
        ],
    )(x_A, segA_i32, x_B, segB_i32)


def _tiny_g_body(sums_ref, cnts_ref, wg_ref, wa_ref, g_ref):
    wg = wg_ref[...]
    wa = wa_ref[...]
    m1 = jnp.dot(wg, wa, preferred_element_type=jnp.float32)
    M = jnp.dot(m1, wg.T, preferred_element_type=jnp.float32)
    sums = sums_ref[...]
    cnts = cnts_ref[...]
    tot = sums[:, 0] + sums[:, 1]
    cnt = jnp.maximum(cnts[:, 0, :, 0] + cnts[:, 1, :, 0], 1.0)
    mean = tot / cnt[:, :, None]
    g_ref[0] = jnp.dot(mean[1], M, preferred_element_type=jnp.float32)
    g_ref[1] = jnp.dot(mean[0], M, preferred_element_type=jnp.float32)


def _pass2_body(xa_ref, sa_ref, xb_ref, sb_ref, g_ref, u_ref, m_ref, d_ref):
    i = pl.program_id(0)

    @pl.when(i == 0)
    def _():
        u_ref[...] = jnp.zeros_like(u_ref)
        m_ref[...] = jnp.full_like(m_ref, NEG)
        d_ref[...] = jnp.zeros_like(d_ref)

    for side, (x_ref, s_ref) in enumerate(((xa_ref, sa_ref), (xb_ref, sb_ref))):
        seg = s_ref[0, 0, :]
        oh = _onehot(seg)
        ohb = seg[:, None] == lax.broadcasted_iota(jnp.int32, (R, S), 1)
        x = x_ref[...]
        g = lax.dot_general(oh, g_ref[side], (((1,), (0,)), ((), ())),
                            preferred_element_type=jnp.float32)
        e = jnp.sum(x * g, axis=1)
        me = jnp.max(jnp.where(ohb, e[:, None], NEG), axis=0)
        m_old = m_ref[side]
        m_new = jnp.maximum(m_old, me)
        m_gath = jnp.sum(oh * m_new[None, :], axis=1)
        w = jnp.exp(e - m_gath)
        scale = jnp.exp(m_old - m_new)
        d_ref[side] = d_ref[side] * scale + jnp.sum(oh * w[:, None], axis=0)
        wx = x * w[:, None]
        u_ref[side] = (u_ref[side] * scale[:, None]
                       + lax.dot_general(oh, wx, (((0,), (0,)), ((), ())),
                                         preferred_element_type=jnp.float32))
        m_ref[side] = m_new


def _final_body(u_ref, d_ref, wg_ref, outa_ref, outb_ref):
    wg = wg_ref[...]
    den = d_ref[...] + 1e-16
    pooled = u_ref[...] / den[:, :, None]
    outa_ref[...] = jnp.dot(pooled[0], wg, preferred_element_type=jnp.float32)
    outb_ref[...] = jnp.dot(pooled[1], wg, preferred_element_type=jnp.float32)


def kernel(x_A, herb_batch_A, x_B, herb_batch_B, W_gnn, W_attn):
    segA_i32 = herb_batch_A.astype(jnp.int32)
    segB_i32 = herb_batch_B.astype(jnp.int32)
    segA = segA_i32.reshape(NB, 1, R)
    segB = segB_i32.reshape(NB, 1, R)

    xspec = pl.BlockSpec((R, D), lambda i: (i, 0))
    sspec = pl.BlockSpec((1, 1, R), lambda i: (i, 0, 0))
    full2 = pl.BlockSpec((2, S, D), lambda i: (0, 0, 0))
    full1 = pl.BlockSpec((2, S), lambda i: (0, 0))

    sums, cnts = _sc_pass1(x_A, segA_i32, x_B, segB_i32)

    G = pl.pallas_call(
        _tiny_g_body,
        out_shape=jax.ShapeDtypeStruct((2, S, D), jnp.float32),
        interpret=_INTERPRET,
    )(sums, cnts, W_gnn, W_attn)

    U, m, d = pl.pallas_call(
        _pass2_body,
        grid=(NB,),
        in_specs=[xspec, sspec, xspec, sspec, full2],
        out_specs=[full2, full1, full1],
        out_shape=[jax.ShapeDtypeStruct((2, S, D), jnp.float32),
                   jax.ShapeDtypeStruct((2, S), jnp.float32),
                   jax.ShapeDtypeStruct((2, S), jnp.float32)],
        compiler_params=pltpu.CompilerParams(
            dimension_semantics=("arbitrary",)),
        interpret=_INTERPRET,
    )(x_A, segA, x_B, segB, G)

    H_A, H_B = pl.pallas_call(
        _final_body,
        out_shape=[jax.ShapeDtypeStruct((S, D), jnp.float32),
                   jax.ShapeDtypeStruct((S, D), jnp.float32)],
        interpret=_INTERPRET,
    )(U, d, W_gnn)
    return (H_A, H_B)


# speedup vs baseline: 1.0243x; 1.0243x over previous
import functools

import jax
import jax.numpy as jnp
from jax import lax
from jax.experimental import pallas as pl
from jax.experimental.pallas import tpu as pltpu
from jax.experimental.pallas import tpu_sc as plsc

N = 100000
D = 128
S = 256
R = 2000
NB = N // R
NEG = -1e30
SW = 64
NST = S // SW

NC, NS, NW = 2, 16, 32
T = 400
SB = 80
NSB = T // SB
NCHUNK = N // T
CPW = (NCHUNK + NW - 1) // NW

_INTERPRET = False


def _onehot(seg, dtype=jnp.float32):
    cols = lax.broadcasted_iota(jnp.int32, (R, S), 1)
    return jnp.where(seg[:, None] == cols, jnp.array(1.0, dtype), jnp.array(0.0, dtype))


def _sc_pass1_body(xa_hbm, sa_hbm, xb_hbm, sb_hbm, sums_out,
                   xbufs, idxbufs, zbuf, semx0, semx1,
                   tab_a, tab_b):
    cid = lax.axis_index("c")
    sid = lax.axis_index("s")
    w = sid * NC + cid
    sems = (semx0, semx1)

    zero = jnp.zeros((16,), jnp.float32)
    for r in range(16):
        for j in range(D // 16):
            zbuf[r, pl.ds(16 * j, 16)] = zero

    row0 = sid * 16
    pltpu.sync_copy(zbuf, tab_a.at[pl.ds(row0, 16)])
    pltpu.sync_copy(zbuf, tab_b.at[pl.ds(row0, 16)])
    plsc.subcore_barrier()

    def _copies(x_hbm, s_hbm, k, b):
        off = k * T
        sem = sems[b]
        cps = [pltpu.make_async_copy(x_hbm.at[pl.ds(off, T)], xbufs.at[b], sem)]
        for i in range(NSB):
            cps.append(pltpu.make_async_copy(
                s_hbm.at[pl.ds(off + i * SB, SB)], idxbufs.at[b, i], sem))
        return cps

    for x_hbm, s_hbm, tab in ((xa_hbm, sa_hbm, tab_a),
                              (xb_hbm, sb_hbm, tab_b)):
        @pl.when(w < NCHUNK)
        def _():
            for cp in _copies(x_hbm, s_hbm, w, 0):
                cp.start()

        for j in range(CPW):
            k = w + NW * j
            b = j % 2
            if j + 1 < CPW:
                k_next = k + NW

                @pl.when(k_next < NCHUNK)
                def _():
                    for cp in _copies(x_hbm, s_hbm, k_next, 1 - b):
                        cp.start()

            @pl.when(k < NCHUNK)
            def _():
                for cp in _copies(x_hbm, s_hbm, k, b):
                    cp.wait()
                for i in range(NSB):
                    pltpu.sync_copy(xbufs.at[b, pl.ds(i * SB, SB)],
                                    tab.at[idxbufs.at[b, i]], add=True)

    plsc.subcore_barrier()

    @pl.when(sid == 0)
    def _():
        pltpu.sync_copy(tab_a, sums_out.at[0, cid])
        pltpu.sync_copy(tab_b, sums_out.at[1, cid])


def _sc_pass1(x_A, segA_i32, x_B, segB_i32):
    return pl.kernel(
        _sc_pass1_body,
        out_type=jax.ShapeDtypeStruct((2, NC, S, D), jnp.float32),
        mesh=plsc.VectorSubcoreMesh(core_axis_name="c", subcore_axis_name="s"),
        scratch_types=[
            pltpu.VMEM((2, T, D), jnp.float32),
            pltpu.VMEM((2, NSB, SB), jnp.int32),
            pltpu.VMEM((16, D), jnp.float32),
            pltpu.SemaphoreType.DMA,
            pltpu.SemaphoreType.DMA,
            pltpu.VMEM_SHARED((S, D), jnp.float32),
            pltpu.VMEM_SHARED((S, D), jnp.float32),
        ],
    )(x_A, segA_i32, x_B, segB_i32)


def _g_body(sa_ref, sb_ref, sums_ref, wg_ref, wa_ref, g_ref, cnt_acc):
    i = pl.program_id(0)

    @pl.when(i == 0)
    def _():
        cnt_acc[...] = jnp.zeros_like(cnt_acc)

    for side, s_ref in enumerate((sa_ref, sb_ref)):
        oh = _onehot(s_ref[0, 0, :])
        cnt_acc[side] += jnp.sum(oh, axis=0)

    @pl.when(i == NB - 1)
    def _():
        wg = wg_ref[...]
        wa = wa_ref[...]
        m1 = jnp.dot(wg, wa, preferred_element_type=jnp.float32)
        M = jnp.dot(m1, wg.T, preferred_element_type=jnp.float32)
        sums = sums_ref[...]
        tot = sums[:, 0] + sums[:, 1]
        cnt = jnp.maximum(cnt_acc[...], 1.0)
        mean = tot / cnt[:, :, None]
        g_ref[0] = jnp.dot(mean[1], M, preferred_element_type=jnp.float32)
        g_ref[1] = jnp.dot(mean[0], M, preferred_element_type=jnp.float32)


def _pass2_body(smin_ref, smax_ref, xa_ref, sa_ref, xb_ref, sb_ref, g_ref,
                u_ref, m_ref, d_ref, g_buf):
    i = pl.program_id(0)

    @pl.when(i == 0)
    def _():
        u_ref[...] = jnp.zeros_like(u_ref)
        m_ref[...] = jnp.full_like(m_ref, NEG)
        d_ref[...] = jnp.zeros_like(d_ref)

    for side, (x_ref, s_ref) in enumerate(((xa_ref, sa_ref), (xb_ref, sb_ref))):
        seg = s_ref[0, 0, :]
        smin = smin_ref[side, i]
        smax = smax_ref[side, i]
        ohb = seg[:, None] == lax.broadcasted_iota(jnp.int32, (R, S), 1)
        oh = jnp.where(ohb, 1.0, 0.0)
        x = x_ref[...]

        g_buf[...] = jnp.zeros_like(g_buf)
        for st in range(NST):
            @pl.when((smin < (st + 1) * SW) & (smax >= st * SW))
            def _():
                g_buf[...] += lax.dot_general(
                    oh[:, st * SW:(st + 1) * SW],
                    g_ref[side, pl.ds(st * SW, SW), :],
                    (((1,), (0,)), ((), ())),
                    preferred_element_type=jnp.float32)

        e = jnp.sum(x * g_buf[...], axis=1)
        me = jnp.max(jnp.where(ohb, e[:, None], NEG), axis=0)
        m_old = m_ref[side]
        m_new = jnp.maximum(m_old, me)
        m_gath = jnp.sum(oh * m_new[None, :], axis=1)
        w = jnp.exp(e - m_gath)
        scale = jnp.exp(m_old - m_new)
        d_ref[side] = d_ref[side] * scale + jnp.sum(oh * w[:, None], axis=0)
        wx = x * w[:, None]
        for st in range(NST):
            @pl.when((smin < (st + 1) * SW) & (smax >= st * SW))
            def _():
                sl = pl.ds(st * SW, SW)
                u_ref[side, sl, :] = (
                    u_ref[side, sl, :] * scale[st * SW:(st + 1) * SW][:, None]
                    + lax.dot_general(oh[:, st * SW:(st + 1) * SW], wx,
                                      (((0,), (0,)), ((), ())),
                                      preferred_element_type=jnp.float32))
        m_ref[side] = m_new


def _final_body(u_ref, d_ref, wg_ref, outa_ref, outb_ref):
    wg = wg_ref[...]
    den = d_ref[...] + 1e-16
    pooled = u_ref[...] / den[:, :, None]
    outa_ref[...] = jnp.dot(pooled[0], wg, preferred_element_type=jnp.float32)
    outb_ref[...] = jnp.dot(pooled[1], wg, preferred_element_type=jnp.float32)


def kernel(x_A, herb_batch_A, x_B, herb_batch_B, W_gnn, W_attn):
    segA_i32 = herb_batch_A.astype(jnp.int32)
    segB_i32 = herb_batch_B.astype(jnp.int32)
    segA = segA_i32.reshape(NB, 1, R)
    segB = segB_i32.reshape(NB, 1, R)
    smin = jnp.stack([segA[:, 0, 0], segB[:, 0, 0]])
    smax = jnp.stack([segA[:, 0, R - 1], segB[:, 0, R - 1]])

    sums = _sc_pass1(x_A, segA_i32, x_B, segB_i32)

    sspec = pl.BlockSpec((1, 1, R), lambda i: (i, 0, 0))
    full_sums = pl.BlockSpec((2, NC, S, D), lambda i: (0, 0, 0, 0))
    wspec = pl.BlockSpec((D, D), lambda i: (0, 0))
    full2 = pl.BlockSpec((2, S, D), lambda i: (0, 0, 0))

    G = pl.pallas_call(
        _g_body,
        grid=(NB,),
        in_specs=[sspec, sspec, full_sums, wspec, wspec],
        out_specs=full2,
        out_shape=jax.ShapeDtypeStruct((2, S, D), jnp.float32),
        scratch_shapes=[pltpu.VMEM((2, S), jnp.float32)],
        compiler_params=pltpu.CompilerParams(
            dimension_semantics=("arbitrary",)),
        interpret=_INTERPRET,
    )(segA, segB, sums, W_gnn, W_attn)

    xspec = pl.BlockSpec((R, D), lambda i, a, b: (i, 0))
    sspec2 = pl.BlockSpec((1, 1, R), lambda i, a, b: (i, 0, 0))
    full2p = pl.BlockSpec((2, S, D), lambda i, a, b: (0, 0, 0))
    full1p = pl.BlockSpec((2, S), lambda i, a, b: (0, 0))

    grid_spec = pltpu.PrefetchScalarGridSpec(
        num_scalar_prefetch=2,
        grid=(NB,),
        in_specs=[xspec, sspec2, xspec, sspec2, full2p],
        out_specs=[full2p, full1p, full1p],
        scratch_shapes=[pltpu.VMEM((R, D), jnp.float32)],
    )
    U, m, d = pl.pallas_call(
        _pass2_body,
        grid_spec=grid_spec,
        out_shape=[jax.ShapeDtypeStruct((2, S, D), jnp.float32),
                   jax.ShapeDtypeStruct((2, S), jnp.float32),
                   jax.ShapeDtypeStruct((2, S), jnp.float32)],
        compiler_params=pltpu.CompilerParams(
            dimension_semantics=("arbitrary",)),
        interpret=_INTERPRET,
    )(smin, smax, x_A, segA, x_B, segB, G)

    H_A, H_B = pl.pallas_call(
        _final_body,
        out_shape=[jax.ShapeDtypeStruct((S, D), jnp.float32),
                   jax.ShapeDtypeStruct((S, D), jnp.float32)],
        interpret=_INTERPRET,
    )(U, d, W_gnn)
    return (H_A, H_B)


# speedup vs baseline: 1.0484x; 1.0235x over previous
import functools

import jax
import jax.numpy as jnp
from jax import lax
from jax.experimental import pallas as pl
from jax.experimental.pallas import tpu as pltpu
from jax.experimental.pallas import tpu_sc as plsc

N = 100000
D = 128
S = 256
R = 2000
NB = N // R
NEG = -1e30
SW = 64
NST = S // SW

NC, NS, NW = 2, 16, 32
T = 400
SB = 80
NSB = T // SB
NCHUNK = N // T
CPW = (NCHUNK + NW - 1) // NW

_INTERPRET = False


def _onehot(seg, dtype=jnp.float32):
    cols = lax.broadcasted_iota(jnp.int32, (R, S), 1)
    return jnp.where(seg[:, None] == cols, jnp.array(1.0, dtype), jnp.array(0.0, dtype))


def _sc_pass1_body(xa_hbm, sa_hbm, xb_hbm, sb_hbm, sums_out,
                   xbufs, idxbufs, zbuf, semx0, semx1,
                   tab_a, tab_b):
    cid = lax.axis_index("c")
    sid = lax.axis_index("s")
    w = sid * NC + cid
    sems = (semx0, semx1)

    zero = jnp.zeros((16,), jnp.float32)
    for r in range(16):
        for j in range(D // 16):
            zbuf[r, pl.ds(16 * j, 16)] = zero

    row0 = sid * 16
    pltpu.sync_copy(zbuf, tab_a.at[pl.ds(row0, 16)])
    pltpu.sync_copy(zbuf, tab_b.at[pl.ds(row0, 16)])
    plsc.subcore_barrier()

    def _copies(x_hbm, s_hbm, k, b):
        off = k * T
        sem = sems[b]
        cps = [pltpu.make_async_copy(x_hbm.at[pl.ds(off, T)], xbufs.at[b], sem)]
        for i in range(NSB):
            cps.append(pltpu.make_async_copy(
                s_hbm.at[pl.ds(off + i * SB, SB)], idxbufs.at[b, i], sem))
        return cps

    for x_hbm, s_hbm, tab in ((xa_hbm, sa_hbm, tab_a),
                              (xb_hbm, sb_hbm, tab_b)):
        @pl.when(w < NCHUNK)
        def _():
            for cp in _copies(x_hbm, s_hbm, w, 0):
                cp.start()

        for j in range(CPW):
            k = w + NW * j
            b = j % 2
            if j + 1 < CPW:
                k_next = k + NW

                @pl.when(k_next < NCHUNK)
                def _():
                    for cp in _copies(x_hbm, s_hbm, k_next, 1 - b):
                        cp.start()

            @pl.when(k < NCHUNK)
            def _():
                for cp in _copies(x_hbm, s_hbm, k, b):
                    cp.wait()
                for i in range(NSB):
                    pltpu.sync_copy(xbufs.at[b, pl.ds(i * SB, SB)],
                                    tab.at[idxbufs.at[b, i]], add=True)

    plsc.subcore_barrier()

    @pl.when(sid == 0)
    def _():
        pltpu.sync_copy(tab_a, sums_out.at[0, cid])
        pltpu.sync_copy(tab_b, sums_out.at[1, cid])


def _sc_pass1(x_A, segA_i32, x_B, segB_i32):
    return pl.kernel(
        _sc_pass1_body,
        out_type=jax.ShapeDtypeStruct((2, NC, S, D), jnp.float32),
        mesh=plsc.VectorSubcoreMesh(core_axis_name="c", subcore_axis_name="s"),
        scratch_types=[
            pltpu.VMEM((2, T, D), jnp.float32),
            pltpu.VMEM((2, NSB, SB), jnp.int32),
            pltpu.VMEM((16, D), jnp.float32),
            pltpu.SemaphoreType.DMA,
            pltpu.SemaphoreType.DMA,
            pltpu.VMEM_SHARED((S, D), jnp.float32),
            pltpu.VMEM_SHARED((S, D), jnp.float32),
        ],
    )(x_A, segA_i32, x_B, segB_i32)


def _g_body(sa_ref, sb_ref, sums_ref, wg_ref, wa_ref, g_ref):
    wg = wg_ref[...]
    wa = wa_ref[...]
    m1 = jnp.dot(wg, wa, preferred_element_type=jnp.float32)
    M = jnp.dot(m1, wg.T, preferred_element_type=jnp.float32)
    cnts = []
    for s_ref in (sa_ref, sb_ref):
        c = jnp.zeros((S,), jnp.float32)
        for j in range(NB):
            c += jnp.sum(_onehot(s_ref[j, 0, :]), axis=0)
        cnts.append(c)
    sums = sums_ref[...]
    tot = sums[:, 0] + sums[:, 1]
    cnt = jnp.maximum(jnp.stack(cnts), 1.0)
    mean = tot / cnt[:, :, None]
    g_ref[0] = jnp.dot(mean[1], M, preferred_element_type=jnp.float32)
    g_ref[1] = jnp.dot(mean[0], M, preferred_element_type=jnp.float32)


def _pass2_body(smin_ref, smax_ref, xa_ref, sa_ref, xb_ref, sb_ref, g_ref,
                u_ref, m_ref, d_ref, g_buf):
    i = pl.program_id(0)

    @pl.when(i == 0)
    def _():
        u_ref[...] = jnp.zeros_like(u_ref)
        m_ref[...] = jnp.full_like(m_ref, NEG)
        d_ref[...] = jnp.zeros_like(d_ref)

    for side, (x_ref, s_ref) in enumerate(((xa_ref, sa_ref), (xb_ref, sb_ref))):
        seg = s_ref[0, 0, :]
        smin = smin_ref[side, i]
        smax = smax_ref[side, i]
        ohb = seg[:, None] == lax.broadcasted_iota(jnp.int32, (R, S), 1)
        oh = jnp.where(ohb, 1.0, 0.0)
        x = x_ref[...]

        g_buf[...] = jnp.zeros_like(g_buf)
        for st in range(NST):
            @pl.when((smin < (st + 1) * SW) & (smax >= st * SW))
            def _():
                g_buf[...] += lax.dot_general(
                    oh[:, st * SW:(st + 1) * SW],
                    g_ref[side, pl.ds(st * SW, SW), :],
                    (((1,), (0,)), ((), ())),
                    preferred_element_type=jnp.float32)

        e = jnp.sum(x * g_buf[...], axis=1)
        me = jnp.max(jnp.where(ohb, e[:, None], NEG), axis=0)
        m_old = m_ref[side]
        m_new = jnp.maximum(m_old, me)
        m_gath = jnp.sum(oh * m_new[None, :], axis=1)
        w = jnp.exp(e - m_gath)
        scale = jnp.exp(m_old - m_new)
        d_ref[side] = d_ref[side] * scale + jnp.sum(oh * w[:, None], axis=0)
        wx = x * w[:, None]
        for st in range(NST):
            @pl.when((smin < (st + 1) * SW) & (smax >= st * SW))
            def _():
                sl = pl.ds(st * SW, SW)
                u_ref[side, sl, :] = (
                    u_ref[side, sl, :] * scale[st * SW:(st + 1) * SW][:, None]
                    + lax.dot_general(oh[:, st * SW:(st + 1) * SW], wx,
                                      (((0,), (0,)), ((), ())),
                                      preferred_element_type=jnp.float32))
        m_ref[side] = m_new


def _final_body(u_ref, d_ref, wg_ref, outa_ref, outb_ref):
    wg = wg_ref[...]
    den = d_ref[...] + 1e-16
    pooled = u_ref[...] / den[:, :, None]
    outa_ref[...] = jnp.dot(pooled[0], wg, preferred_element_type=jnp.float32)
    outb_ref[...] = jnp.dot(pooled[1], wg, preferred_element_type=jnp.float32)


def kernel(x_A, herb_batch_A, x_B, herb_batch_B, W_gnn, W_attn):
    segA_i32 = herb_batch_A.astype(jnp.int32)
    segB_i32 = herb_batch_B.astype(jnp.int32)
    segA = segA_i32.reshape(NB, 1, R)
    segB = segB_i32.reshape(NB, 1, R)
    smin = jnp.stack([segA[:, 0, 0], segB[:, 0, 0]])
    smax = jnp.stack([segA[:, 0, R - 1], segB[:, 0, R - 1]])

    sums = _sc_pass1(x_A, segA_i32, x_B, segB_i32)

    G = pl.pallas_call(
        _g_body,
        out_shape=jax.ShapeDtypeStruct((2, S, D), jnp.float32),
        interpret=_INTERPRET,
    )(segA, segB, sums, W_gnn, W_attn)

    xspec = pl.BlockSpec((R, D), lambda i, a, b: (i, 0))
    sspec2 = pl.BlockSpec((1, 1, R), lambda i, a, b: (i, 0, 0))
    full2p = pl.BlockSpec((2, S, D), lambda i, a, b: (0, 0, 0))
    full1p = pl.BlockSpec((2, S), lambda i, a, b: (0, 0))

    grid_spec = pltpu.PrefetchScalarGridSpec(
        num_scalar_prefetch=2,
        grid=(NB,),
        in_specs=[xspec, sspec2, xspec, sspec2, full2p],
        out_specs=[full2p, full1p, full1p],
        scratch_shapes=[pltpu.VMEM((R, D), jnp.float32)],
    )
    U, m, d = pl.pallas_call(
        _pass2_body,
        grid_spec=grid_spec,
        out_shape=[jax.ShapeDtypeStruct((2, S, D), jnp.float32),
                   jax.ShapeDtypeStruct((2, S), jnp.float32),
                   jax.ShapeDtypeStruct((2, S), jnp.float32)],
        compiler_params=pltpu.CompilerParams(
            dimension_semantics=("arbitrary",)),
        interpret=_INTERPRET,
    )(smin, smax, x_A, segA, x_B, segB, G)

    H_A, H_B = pl.pallas_call(
        _final_body,
        out_shape=[jax.ShapeDtypeStruct((S, D), jnp.float32),
                   jax.ShapeDtypeStruct((S, D), jnp.float32)],
        interpret=_INTERPRET,
    )(U, d, W_gnn)
    return (H_A, H_B)
